# baseline (device time: 139868 ns/iter reference)
import jax
import jax.numpy as jnp
from jax import lax
from jax.experimental import pallas as pl
from jax.experimental.pallas import tpu as pltpu

N_DEV = 8
U = 1152
V1 = 512
V2 = 384


def kernel(x, w_mat):
    m, k_per = x.shape
    _, n = w_mat.shape
    m_chunk = m // N_DEV

    def body(x_ref, w_ref, out_ref,
             s1cw, s1ccw, t1cw, t1ccw, s1z_recv,
             z_send, z_recv, s2cw, s2ccw,
             ss1cw, rs1cw, ss1ccw, rs1ccw,
             s1z_ssem, s1z_rsem, z_ssem, z_rsem,
             ss2cw, rs2cw, ss2ccw, rs2ccw):
        my = lax.axis_index("i")
        plane = my // 4
        k = lax.rem(my, 4)
        right = plane * 4 + lax.rem(k + 1, 4)
        left = plane * 4 + lax.rem(k + 3, 4)
        partner = lax.rem(my + 4, N_DEV)

        barrier_sem = pltpu.get_barrier_semaphore()
        for nbr in (right, left, partner):
            pl.semaphore_signal(
                barrier_sem, inc=1,
                device_id=(nbr,), device_id_type=pl.DeviceIdType.MESH,
            )
        pl.semaphore_wait(barrier_sem, 3)

        def partial(chunk_id, lo, width):
            rows = x_ref[pl.ds(chunk_id * m_chunk, m_chunk), :]
            return jnp.dot(rows, w_ref[:, lo:lo + width],
                           preferred_element_type=jnp.float32)

        def ring_rdma(buf, ssem, rsem, h, target):
            return pltpu.make_async_remote_copy(
                src_ref=buf.at[h % 3], dst_ref=buf.at[(h + 1) % 3],
                send_sem=ssem.at[h % 3], recv_sem=rsem.at[(h + 1) % 3],
                device_id=(target,), device_id_type=pl.DeviceIdType.MESH,
            )

        def own_chunk(j):
            return plane * 4 + j
        def other_chunk(j):
            return (1 - plane) * 4 + j

        s1cw[0, :, :] = partial(own_chunk(lax.rem(k + 3, 4)), 0, U)
        ring_rdma(s1cw, ss1cw, rs1cw, 0, right).start()
        s1ccw[0, :, :] = partial(other_chunk(lax.rem(k + 1, 4)), 0, U)
        ring_rdma(s1ccw, ss1ccw, rs1ccw, 0, left).start()

        j_seq = [lax.rem(k + 3, 4), lax.rem(k + 1, 4),
                 lax.rem(k + 2, 4), k]

        def z_rdma(idx):
            return pltpu.make_async_remote_copy(
                src_ref=z_send.at[idx], dst_ref=z_recv.at[idx],
                send_sem=z_ssem.at[idx], recv_sem=z_rsem.at[idx],
                device_id=(partner,), device_id_type=pl.DeviceIdType.MESH,
            )

        for idx in range(4):
            z_send[idx, :, :] = partial(other_chunk(j_seq[idx]), U, n - U)
            z_rdma(idx).start()

        def comb_cw(j, idx):
            return partial(own_chunk(j), U, V1) + z_recv[idx, :, :V1]
        def comb_ccw(j, idx):
            return partial(own_chunk(j), U + V1, V2) + z_recv[idx, :, V1:]

        t1cw[:, :] = partial(own_chunk(lax.rem(k + 2, 4)), 0, U)
        t1ccw[:, :] = partial(other_chunk(lax.rem(k + 2, 4)), 0, U)
        z_rdma(0).wait_recv()
        s2cw[0, :, :] = comb_cw(lax.rem(k + 3, 4), 0)
        ring_rdma(s2cw, ss2cw, rs2cw, 0, right).start()

        for h in range(3):
            if h > 0:
                t1cw[:, :] = partial(own_chunk(lax.rem(k - h + 6, 4)), 0, U)
                t1ccw[:, :] = partial(other_chunk(lax.rem(k + h + 2, 4)), 0, U)
            r = (h + 1) % 3
            ring_rdma(s1cw, ss1cw, rs1cw, h, right).wait()
            s1cw[r, :, :] = s1cw[r, :, :] + t1cw[:, :]
            if h < 2:
                ring_rdma(s1cw, ss1cw, rs1cw, h + 1, right).start()
            ring_rdma(s1ccw, ss1ccw, rs1ccw, h, left).wait()
            s1ccw[r, :, :] = s1ccw[r, :, :] + t1ccw[:, :]
            if h < 2:
                ring_rdma(s1ccw, ss1ccw, rs1ccw, h + 1, left).start()
            if h == 1:
                z_rdma(2).wait_recv()
                ring_rdma(s2cw, ss2cw, rs2cw, 0, right).wait()
                s2cw[1, :, :] = s2cw[1, :, :] + comb_cw(lax.rem(k + 2, 4), 2)
                ring_rdma(s2cw, ss2cw, rs2cw, 1, right).start()
                z_rdma(1).wait_recv()
                s2ccw[0, :, :] = comb_ccw(lax.rem(k + 1, 4), 1)
                ring_rdma(s2ccw, ss2ccw, rs2ccw, 0, left).start()

        s1z = pltpu.make_async_remote_copy(
            src_ref=s1ccw.at[0], dst_ref=s1z_recv,
            send_sem=s1z_ssem, recv_sem=s1z_rsem,
            device_id=(partner,), device_id_type=pl.DeviceIdType.MESH,
        )
        s1z.start()

        ring_rdma(s2cw, ss2cw, rs2cw, 1, right).wait()
        s2cw[2, :, :] = s2cw[2, :, :] + comb_cw(lax.rem(k + 1, 4), 1)
        ring_rdma(s2cw, ss2cw, rs2cw, 2, right).start()
        ring_rdma(s2ccw, ss2ccw, rs2ccw, 0, left).wait()
        s2ccw[1, :, :] = s2ccw[1, :, :] + comb_ccw(lax.rem(k + 2, 4), 2)
        ring_rdma(s2ccw, ss2ccw, rs2ccw, 1, left).start()
        ring_rdma(s2ccw, ss2ccw, rs2ccw, 1, left).wait()
        s2ccw[2, :, :] = s2ccw[2, :, :] + comb_ccw(lax.rem(k + 3, 4), 0)
        ring_rdma(s2ccw, ss2ccw, rs2ccw, 2, left).start()
        z_rdma(3).wait_recv()
        ring_rdma(s2cw, ss2cw, rs2cw, 2, right).wait()
        out_ref[:, U:U + V1] = s2cw[0, :, :] + comb_cw(k, 3)
        ring_rdma(s2ccw, ss2ccw, rs2ccw, 2, left).wait()
        out_ref[:, U + V1:] = s2ccw[0, :, :] + comb_ccw(k, 3)

        s1z.wait_recv()
        out_ref[:, :U] = s1cw[0, :, :] + s1z_recv[:, :]

        s1z.wait_send()
        for idx in range(4):
            z_rdma(idx).wait_send()

    return pl.pallas_call(
        body,
        out_shape=jax.ShapeDtypeStruct((m_chunk, n), jnp.float32),
        in_specs=[
            pl.BlockSpec(memory_space=pltpu.VMEM),
            pl.BlockSpec(memory_space=pltpu.VMEM),
        ],
        out_specs=pl.BlockSpec(memory_space=pltpu.VMEM),
        scratch_shapes=[
            pltpu.VMEM((3, m_chunk, U), jnp.float32),
            pltpu.VMEM((3, m_chunk, U), jnp.float32),
            pltpu.VMEM((m_chunk, U), jnp.float32),
            pltpu.VMEM((m_chunk, U), jnp.float32),
            pltpu.VMEM((m_chunk, U), jnp.float32),
            pltpu.VMEM((4, m_chunk, n - U), jnp.float32),
            pltpu.VMEM((4, m_chunk, n - U), jnp.float32),
            pltpu.VMEM((3, m_chunk, V1), jnp.float32),
            pltpu.VMEM((3, m_chunk, V2), jnp.float32),
            pltpu.SemaphoreType.DMA((3,)),
            pltpu.SemaphoreType.DMA((3,)),
            pltpu.SemaphoreType.DMA((3,)),
            pltpu.SemaphoreType.DMA((3,)),
            pltpu.SemaphoreType.DMA,
            pltpu.SemaphoreType.DMA,
            pltpu.SemaphoreType.DMA((4,)),
            pltpu.SemaphoreType.DMA((4,)),
            pltpu.SemaphoreType.DMA((3,)),
            pltpu.SemaphoreType.DMA((3,)),
            pltpu.SemaphoreType.DMA((3,)),
            pltpu.SemaphoreType.DMA((3,)),
        ],
        compiler_params=pltpu.CompilerParams(
            collective_id=0,
            vmem_limit_bytes=100 * 1024 * 1024,
        ),
    )(x, w_mat)


# device time: 132499 ns/iter; 1.0556x vs baseline; 1.0556x over previous
import jax
import jax.numpy as jnp
from jax import lax
from jax.experimental import pallas as pl
from jax.experimental.pallas import tpu as pltpu

N_DEV = 8
U = 1024
NSUB = 4
SUB = 256


def kernel(x, w_mat):
    m, k_per = x.shape
    _, n = w_mat.shape
    m_chunk = m // N_DEV

    def body(x_ref, w_ref, out_ref,
             s1cw, s1ccw, t1cw, t1ccw, s1z_recv,
             z_send, z_recv, s2a, s2b, s2c, s2d,
             ss1cw, rs1cw, ss1ccw, rs1ccw,
             s1z_ssem, s1z_rsem, z_ssem, z_rsem,
             *s2sems):
        my = lax.axis_index("i")
        plane = my // 4
        k = lax.rem(my, 4)
        right = plane * 4 + lax.rem(k + 1, 4)
        left = plane * 4 + lax.rem(k + 3, 4)
        partner = lax.rem(my + 4, N_DEV)

        barrier_sem = pltpu.get_barrier_semaphore()
        for nbr in (right, left, partner):
            pl.semaphore_signal(
                barrier_sem, inc=1,
                device_id=(nbr,), device_id_type=pl.DeviceIdType.MESH,
            )
        pl.semaphore_wait(barrier_sem, 3)

        def partial(chunk_id, lo, width):
            rows = x_ref[pl.ds(chunk_id * m_chunk, m_chunk), :]
            return jnp.dot(rows, w_ref[:, lo:lo + width],
                           preferred_element_type=jnp.float32)

        def ring_rdma(buf, ssem, rsem, h, target):
            return pltpu.make_async_remote_copy(
                src_ref=buf.at[h % 3], dst_ref=buf.at[(h + 1) % 3],
                send_sem=ssem.at[h % 3], recv_sem=rsem.at[(h + 1) % 3],
                device_id=(target,), device_id_type=pl.DeviceIdType.MESH,
            )

        def own_chunk(j):
            return plane * 4 + j
        def other_chunk(j):
            return (1 - plane) * 4 + j

        j_seq = [lax.rem(k + 3, 4), lax.rem(k + 1, 4),
                 lax.rem(k + 2, 4), k]

        def z_rdma(idx):
            return pltpu.make_async_remote_copy(
                src_ref=z_send.at[idx], dst_ref=z_recv.at[idx],
                send_sem=z_ssem.at[idx], recv_sem=z_rsem.at[idx],
                device_id=(partner,), device_id_type=pl.DeviceIdType.MESH,
            )

        for idx in range(4):
            z_send[idx, :, :] = partial(other_chunk(j_seq[idx]), U, n - U)
            z_rdma(idx).start()

        s1cw[0, :, :] = partial(own_chunk(lax.rem(k + 3, 4)), 0, U)
        ring_rdma(s1cw, ss1cw, rs1cw, 0, right).start()
        s1ccw[0, :, :] = partial(other_chunk(lax.rem(k + 1, 4)), 0, U)
        ring_rdma(s1ccw, ss1ccw, rs1ccw, 0, left).start()

        for h in range(3):
            t1cw[:, :] = partial(own_chunk(lax.rem(k - h + 6, 4)), 0, U)
            t1ccw[:, :] = partial(other_chunk(lax.rem(k + h + 2, 4)), 0, U)
            r = (h + 1) % 3
            ring_rdma(s1cw, ss1cw, rs1cw, h, right).wait()
            s1cw[r, :, :] = s1cw[r, :, :] + t1cw[:, :]
            if h < 2:
                ring_rdma(s1cw, ss1cw, rs1cw, h + 1, right).start()
            ring_rdma(s1ccw, ss1ccw, rs1ccw, h, left).wait()
            s1ccw[r, :, :] = s1ccw[r, :, :] + t1ccw[:, :]
            if h < 2:
                ring_rdma(s1ccw, ss1ccw, rs1ccw, h + 1, left).start()

        s1z = pltpu.make_async_remote_copy(
            src_ref=s1ccw.at[0], dst_ref=s1z_recv,
            send_sem=s1z_ssem, recv_sem=s1z_rsem,
            device_id=(partner,), device_id_type=pl.DeviceIdType.MESH,
        )
        s1z.start()

        streams = [
            (s2a, s2sems[0], s2sems[1], 0, True),
            (s2b, s2sems[2], s2sems[3], SUB, True),
            (s2c, s2sems[4], s2sems[5], 2 * SUB, False),
            (s2d, s2sems[6], s2sems[7], 3 * SUB, False),
        ]

        def comb(j, idx, off):
            return (partial(own_chunk(j), U + off, SUB)
                    + z_recv[idx, :, off:off + SUB])

        z_rdma(0).wait_recv()
        z_rdma(1).wait_recv()
        for buf, ssem, rsem, off, cw in streams:
            j0 = lax.rem(k + 3, 4) if cw else lax.rem(k + 1, 4)
            buf[0, :, :] = comb(j0, 0 if cw else 1, off)
            ring_rdma(buf, ssem, rsem, 0, right if cw else left).start()

        for h in range(3):
            if h == 0:
                z_rdma(2).wait_recv()
            if h == 2:
                z_rdma(3).wait_recv()
            for buf, ssem, rsem, off, cw in streams:
                if cw:
                    j, idx = [(lax.rem(k + 2, 4), 2),
                              (lax.rem(k + 1, 4), 1), (k, 3)][h]
                else:
                    j, idx = [(lax.rem(k + 2, 4), 2),
                              (lax.rem(k + 3, 4), 0), (k, 3)][h]
                r = (h + 1) % 3
                ring_rdma(buf, ssem, rsem, h, right if cw else left).wait()
                if h < 2:
                    buf[r, :, :] = buf[r, :, :] + comb(j, idx, off)
                    ring_rdma(buf, ssem, rsem, h + 1,
                              right if cw else left).start()
                else:
                    out_ref[:, U + off:U + off + SUB] = (
                        buf[r, :, :] + comb(j, idx, off))

        s1z.wait_recv()
        out_ref[:, :U] = s1cw[0, :, :] + s1z_recv[:, :]

        s1z.wait_send()
        for idx in range(4):
            z_rdma(idx).wait_send()

    return pl.pallas_call(
        body,
        out_shape=jax.ShapeDtypeStruct((m_chunk, n), jnp.float32),
        in_specs=[
            pl.BlockSpec(memory_space=pltpu.VMEM),
            pl.BlockSpec(memory_space=pltpu.VMEM),
        ],
        out_specs=pl.BlockSpec(memory_space=pltpu.VMEM),
        scratch_shapes=[
            pltpu.VMEM((3, m_chunk, U), jnp.float32),
            pltpu.VMEM((3, m_chunk, U), jnp.float32),
            pltpu.VMEM((m_chunk, U), jnp.float32),
            pltpu.VMEM((m_chunk, U), jnp.float32),
            pltpu.VMEM((m_chunk, U), jnp.float32),
            pltpu.VMEM((4, m_chunk, n - U), jnp.float32),
            pltpu.VMEM((4, m_chunk, n - U), jnp.float32),
            pltpu.VMEM((3, m_chunk, SUB), jnp.float32),
            pltpu.VMEM((3, m_chunk, SUB), jnp.float32),
            pltpu.VMEM((3, m_chunk, SUB), jnp.float32),
            pltpu.VMEM((3, m_chunk, SUB), jnp.float32),
            pltpu.SemaphoreType.DMA((3,)),
            pltpu.SemaphoreType.DMA((3,)),
            pltpu.SemaphoreType.DMA((3,)),
            pltpu.SemaphoreType.DMA((3,)),
            pltpu.SemaphoreType.DMA,
            pltpu.SemaphoreType.DMA,
            pltpu.SemaphoreType.DMA((4,)),
            pltpu.SemaphoreType.DMA((4,)),
        ] + [pltpu.SemaphoreType.DMA((3,)) for _ in range(8)],
        compiler_params=pltpu.CompilerParams(
            collective_id=0,
            vmem_limit_bytes=100 * 1024 * 1024,
        ),
    )(x, w_mat)


# device time: 130167 ns/iter; 1.0745x vs baseline; 1.0179x over previous
import jax
import jax.numpy as jnp
from jax import lax
from jax.experimental import pallas as pl
from jax.experimental.pallas import tpu as pltpu

N_DEV = 8
U = 1024
NSUB = 4
SUB = 256


def kernel(x, w_mat):
    m, k_per = x.shape
    _, n = w_mat.shape
    m_chunk = m // N_DEV

    def body(x_ref, w_ref, out_ref,
             s1cw, s1ccw, t1cw, t1ccw, s1z_recv,
             z_send, z_recv, s2a, s2b, s2c, s2d,
             ss1cw, rs1cw, ss1ccw, rs1ccw,
             s1z_ssem, s1z_rsem, z_ssem, z_rsem,
             *s2sems):
        my = lax.axis_index("i")
        plane = my // 4
        k = lax.rem(my, 4)
        right = plane * 4 + lax.rem(k + 1, 4)
        left = plane * 4 + lax.rem(k + 3, 4)
        partner = lax.rem(my + 4, N_DEV)

        barrier_sem = pltpu.get_barrier_semaphore()
        for nbr in (right, left, partner):
            pl.semaphore_signal(
                barrier_sem, inc=1,
                device_id=(nbr,), device_id_type=pl.DeviceIdType.MESH,
            )
        pl.semaphore_wait(barrier_sem, 3)

        def partial(chunk_id, lo, width):
            rows = x_ref[pl.ds(chunk_id * m_chunk, m_chunk), :]
            return jnp.dot(rows, w_ref[:, lo:lo + width],
                           preferred_element_type=jnp.float32)

        def ring_rdma(buf, ssem, rsem, h, target):
            return pltpu.make_async_remote_copy(
                src_ref=buf.at[h % 3], dst_ref=buf.at[(h + 1) % 3],
                send_sem=ssem.at[h % 3], recv_sem=rsem.at[(h + 1) % 3],
                device_id=(target,), device_id_type=pl.DeviceIdType.MESH,
            )

        def own_chunk(j):
            return plane * 4 + j
        def other_chunk(j):
            return (1 - plane) * 4 + j

        j_seq = [lax.rem(k + 3, 4), lax.rem(k + 1, 4),
                 lax.rem(k + 2, 4), k]

        def z_rdma(idx):
            return pltpu.make_async_remote_copy(
                src_ref=z_send.at[idx], dst_ref=z_recv.at[idx],
                send_sem=z_ssem.at[idx], recv_sem=z_rsem.at[idx],
                device_id=(partner,), device_id_type=pl.DeviceIdType.MESH,
            )

        z_send[0, :, :] = partial(other_chunk(j_seq[0]), U, n - U)
        z_rdma(0).start()

        s1cw[0, :, :] = partial(own_chunk(lax.rem(k + 3, 4)), 0, U)
        ring_rdma(s1cw, ss1cw, rs1cw, 0, right).start()
        s1ccw[0, :, :] = partial(other_chunk(lax.rem(k + 1, 4)), 0, U)
        ring_rdma(s1ccw, ss1ccw, rs1ccw, 0, left).start()

        for idx in range(1, 4):
            z_send[idx, :, :] = partial(other_chunk(j_seq[idx]), U, n - U)
            z_rdma(idx).start()

        for h in range(3):
            t1cw[:, :] = partial(own_chunk(lax.rem(k - h + 6, 4)), 0, U)
            t1ccw[:, :] = partial(other_chunk(lax.rem(k + h + 2, 4)), 0, U)
            r = (h + 1) % 3
            ring_rdma(s1cw, ss1cw, rs1cw, h, right).wait()
            s1cw[r, :, :] = s1cw[r, :, :] + t1cw[:, :]
            if h < 2:
                ring_rdma(s1cw, ss1cw, rs1cw, h + 1, right).start()
            ring_rdma(s1ccw, ss1ccw, rs1ccw, h, left).wait()
            s1ccw[r, :, :] = s1ccw[r, :, :] + t1ccw[:, :]
            if h < 2:
                ring_rdma(s1ccw, ss1ccw, rs1ccw, h + 1, left).start()

        s1z = pltpu.make_async_remote_copy(
            src_ref=s1ccw.at[0], dst_ref=s1z_recv,
            send_sem=s1z_ssem, recv_sem=s1z_rsem,
            device_id=(partner,), device_id_type=pl.DeviceIdType.MESH,
        )
        s1z.start()

        streams = [
            (s2a, s2sems[0], s2sems[1], 0, True),
            (s2b, s2sems[2], s2sems[3], SUB, True),
            (s2c, s2sems[4], s2sems[5], 2 * SUB, False),
            (s2d, s2sems[6], s2sems[7], 3 * SUB, False),
        ]

        def comb(j, idx, off):
            return (partial(own_chunk(j), U + off, SUB)
                    + z_recv[idx, :, off:off + SUB])

        z_rdma(0).wait_recv()
        z_rdma(1).wait_recv()
        for buf, ssem, rsem, off, cw in streams:
            j0 = lax.rem(k + 3, 4) if cw else lax.rem(k + 1, 4)
            buf[0, :, :] = comb(j0, 0 if cw else 1, off)
            ring_rdma(buf, ssem, rsem, 0, right if cw else left).start()

        for h in range(3):
            if h == 0:
                z_rdma(2).wait_recv()
            if h == 2:
                z_rdma(3).wait_recv()
            for buf, ssem, rsem, off, cw in streams:
                if cw:
                    j, idx = [(lax.rem(k + 2, 4), 2),
                              (lax.rem(k + 1, 4), 1), (k, 3)][h]
                else:
                    j, idx = [(lax.rem(k + 2, 4), 2),
                              (lax.rem(k + 3, 4), 0), (k, 3)][h]
                r = (h + 1) % 3
                ring_rdma(buf, ssem, rsem, h, right if cw else left).wait()
                if h < 2:
                    buf[r, :, :] = buf[r, :, :] + comb(j, idx, off)
                    ring_rdma(buf, ssem, rsem, h + 1,
                              right if cw else left).start()
                else:
                    out_ref[:, U + off:U + off + SUB] = (
                        buf[r, :, :] + comb(j, idx, off))

        s1z.wait_recv()
        out_ref[:, :U] = s1cw[0, :, :] + s1z_recv[:, :]

        s1z.wait_send()
        for idx in range(4):
            z_rdma(idx).wait_send()

    return pl.pallas_call(
        body,
        out_shape=jax.ShapeDtypeStruct((m_chunk, n), jnp.float32),
        in_specs=[
            pl.BlockSpec(memory_space=pltpu.VMEM),
            pl.BlockSpec(memory_space=pltpu.VMEM),
        ],
        out_specs=pl.BlockSpec(memory_space=pltpu.VMEM),
        scratch_shapes=[
            pltpu.VMEM((3, m_chunk, U), jnp.float32),
            pltpu.VMEM((3, m_chunk, U), jnp.float32),
            pltpu.VMEM((m_chunk, U), jnp.float32),
            pltpu.VMEM((m_chunk, U), jnp.float32),
            pltpu.VMEM((m_chunk, U), jnp.float32),
            pltpu.VMEM((4, m_chunk, n - U), jnp.float32),
            pltpu.VMEM((4, m_chunk, n - U), jnp.float32),
            pltpu.VMEM((3, m_chunk, SUB), jnp.float32),
            pltpu.VMEM((3, m_chunk, SUB), jnp.float32),
            pltpu.VMEM((3, m_chunk, SUB), jnp.float32),
            pltpu.VMEM((3, m_chunk, SUB), jnp.float32),
            pltpu.SemaphoreType.DMA((3,)),
            pltpu.SemaphoreType.DMA((3,)),
            pltpu.SemaphoreType.DMA((3,)),
            pltpu.SemaphoreType.DMA((3,)),
            pltpu.SemaphoreType.DMA,
            pltpu.SemaphoreType.DMA,
            pltpu.SemaphoreType.DMA((4,)),
            pltpu.SemaphoreType.DMA((4,)),
        ] + [pltpu.SemaphoreType.DMA((3,)) for _ in range(8)],
        compiler_params=pltpu.CompilerParams(
            collective_id=0,
            vmem_limit_bytes=100 * 1024 * 1024,
        ),
    )(x, w_mat)
